# baseline (speedup 1.0000x reference)
import functools

import jax
import jax.numpy as jnp
from jax import lax
from jax.experimental import pallas as pl
from jax.experimental.pallas import tpu as pltpu
from jax.experimental.pallas import tpu_sc as plsc

B = 16384
F = 26
V = 100000
D = 16
ND = 13
H1 = 256
H2 = 128
FD = F * D

_NC = 2
_NS = 16
_NW = _NC * _NS
_NTASK = FD + F
_NROUND = -(-_NTASK // _NW)


def _make_sc_gather(nf):
    ntask = nf * D + nf
    nround = -(-ntask // _NW)

    def body(emb_hbm, lin_hbm, idx_hbm, fen_out, lin_out,
             row_v, idx_v, res_v):
        wid = lax.axis_index("s") * _NC + lax.axis_index("c")
        nd = nf * D

        for j in range(nround):
            t = jnp.minimum(wid + _NW * j, ntask - 1)
            is_lin = t >= nd
            f = jnp.where(is_lin, t - nd, t // D)
            pltpu.sync_copy(idx_hbm.at[pl.ds(f * B, B)], idx_v)

            @pl.when(jnp.logical_not(is_lin))
            def _le():
                pltpu.sync_copy(emb_hbm.at[pl.ds(t * V, V)], row_v)

            @pl.when(is_lin)
            def _ll():
                pltpu.sync_copy(lin_hbm.at[pl.ds((t - nd) * V, V)], row_v)

            for cix in range(B // _CH):
                @plsc.parallel_loop(0, _CH // 16, unroll=8)
                def _gather(i):
                    off = i * 16
                    ii = idx_v[pl.ds(cix * _CH + off, 16)]
                    res_v[pl.ds(off, 16)] = plsc.load_gather(row_v, [ii])

                @pl.when(jnp.logical_not(is_lin))
                def _of():
                    pltpu.sync_copy(
                        res_v, fen_out.at[pl.ds(t * B + cix * _CH, _CH)])

                @pl.when(is_lin)
                def _ol():
                    pltpu.sync_copy(
                        res_v,
                        lin_out.at[pl.ds((t - nd) * B + cix * _CH, _CH)])

    call = pl.kernel(
        body,
        out_type=(
            jax.ShapeDtypeStruct((nf * D * B,), jnp.float32),
            jax.ShapeDtypeStruct((nf * B,), jnp.float32),
        ),
        mesh=plsc.VectorSubcoreMesh(core_axis_name="c", subcore_axis_name="s"),
        scratch_types=[
            pltpu.VMEM((V,), jnp.float32),
            pltpu.VMEM((B,), jnp.int32),
            pltpu.VMEM((_CH,), jnp.float32),
        ],
        compiler_params=pltpu.CompilerParams(
            use_tc_tiling_on_sc=False, needs_layout_passes=False),
    )
    return call


_CH = 4096
_G = (13, 13)


_BB = 2048


def _tc_body(fen_a_ref, fen_b_ref, lin_a_ref, lin_b_ref, den_ref,
             w1t_ref, b1_ref, w2t_ref, b2_ref,
             pt_ref, e_ref, s_ref, dw_ref, bias_ref, out_ref):
    x = jnp.concatenate([fen_a_ref[...], fen_b_ref[...]], axis=0)
    lin_ref_all = jnp.concatenate([lin_a_ref[...], lin_b_ref[...]], axis=0)
    h = jnp.dot(w1t_ref[...], x, preferred_element_type=jnp.float32)
    h = jnp.maximum(h + b1_ref[...], 0.0)
    h = jnp.dot(w2t_ref[...], h, preferred_element_type=jnp.float32)
    h = jnp.maximum(h + b2_ref[...], 0.0)
    logits = jnp.dot(pt_ref[...], h, preferred_element_type=jnp.float32)
    m = jnp.max(logits, axis=0, keepdims=True)
    e = jnp.exp(logits - m)
    mx = (float(F) / jnp.sum(e, axis=0, keepdims=True)) * e
    mx_exp = jnp.dot(e_ref[...], mx, preferred_element_type=jnp.float32)
    v = mx_exp * x
    sv = jnp.dot(s_ref[...], v, preferred_element_type=jnp.float32)
    fm = 0.5 * (jnp.sum(sv * sv, axis=0) - jnp.sum(v * v, axis=0))
    sp = jnp.sum(lin_ref_all * mx, axis=0)
    dn = jnp.sum(den_ref[...] * dw_ref[...], axis=0)
    out_ref[...] = fm + sp + dn + bias_ref[0, 0]


def kernel(sparse, dense, emb, lin_w, dense_w, W1, b1, W2, b2, P, bias):
    emb_t3 = emb.transpose(0, 2, 1)
    idx_t = sparse.T

    fens, lins = [], []
    f0 = 0
    for nf in _G:
        emb_g = emb_t3[f0:f0 + nf].reshape(nf * D * V)
        lin_g = lin_w[f0:f0 + nf].reshape(nf * V)
        idx_g = idx_t[f0:f0 + nf].reshape(nf * B)
        fen_f, lin_f = _make_sc_gather(nf)(emb_g, lin_g, idx_g)
        fens.append(fen_f.reshape(nf * D, B))
        lins.append(lin_f.reshape(nf, B))
        f0 += nf
    dense_t = dense.T

    expand_t = jnp.repeat(jnp.eye(F, dtype=jnp.float32), D, axis=0)
    reduce_t = jnp.tile(jnp.eye(D, dtype=jnp.float32), (1, F))

    na, nb = _G
    out = pl.pallas_call(
        _tc_body,
        grid=(B // _BB,),
        in_specs=[
            pl.BlockSpec((na * D, _BB), lambda i: (0, i)),
            pl.BlockSpec((nb * D, _BB), lambda i: (0, i)),
            pl.BlockSpec((na, _BB), lambda i: (0, i)),
            pl.BlockSpec((nb, _BB), lambda i: (0, i)),
            pl.BlockSpec((ND, _BB), lambda i: (0, i)),
            pl.BlockSpec((H1, FD), lambda i: (0, 0)),
            pl.BlockSpec((H1, 1), lambda i: (0, 0)),
            pl.BlockSpec((H2, H1), lambda i: (0, 0)),
            pl.BlockSpec((H2, 1), lambda i: (0, 0)),
            pl.BlockSpec((F, H2), lambda i: (0, 0)),
            pl.BlockSpec((FD, F), lambda i: (0, 0)),
            pl.BlockSpec((D, FD), lambda i: (0, 0)),
            pl.BlockSpec((ND, 1), lambda i: (0, 0)),
            pl.BlockSpec((1, 1), lambda i: (0, 0)),
        ],
        out_specs=pl.BlockSpec((_BB,), lambda i: (i,)),
        out_shape=jax.ShapeDtypeStruct((B,), jnp.float32),
    )(fens[0], fens[1], lins[0], lins[1], dense_t,
      W1.T, b1.reshape(H1, 1), W2.T, b2.reshape(H2, 1),
      P.T, expand_t, reduce_t, dense_w.reshape(ND, 1), bias.reshape(1, 1))
    return out


# speedup vs baseline: 1.1753x; 1.1753x over previous
import functools

import jax
import jax.numpy as jnp
from jax import lax
from jax.experimental import pallas as pl
from jax.experimental.pallas import tpu as pltpu
from jax.experimental.pallas import tpu_sc as plsc

B = 16384
F = 26
V = 100000
D = 16
ND = 13
H1 = 256
H2 = 128
FD = F * D

_NC = 2
_NS = 16
_NW = _NC * _NS
_NTASK = FD + F
_NROUND = -(-_NTASK // _NW)


def _make_sc_gather(nf):
    ntask = nf * D + nf
    nround = -(-ntask // _NW)

    def body(emb_hbm, lin_hbm, idx_hbm, fen_out, lin_out,
             row_v, idx_v, res_v):
        wid = lax.axis_index("s") * _NC + lax.axis_index("c")
        nd = nf * D

        for j in range(nround):
            t = jnp.minimum(wid + _NW * j, ntask - 1)
            is_lin = t >= nd
            f = jnp.where(is_lin, t - nd, t // D)
            pltpu.sync_copy(idx_hbm.at[pl.ds(f * B, B)], idx_v)

            @pl.when(jnp.logical_not(is_lin))
            def _le():
                pltpu.sync_copy(emb_hbm.at[pl.ds(t * V, V)], row_v)

            @pl.when(is_lin)
            def _ll():
                pltpu.sync_copy(lin_hbm.at[pl.ds((t - nd) * V, V)], row_v)

            for cix in range(B // _CH):
                @plsc.parallel_loop(0, _CH // 16, unroll=8)
                def _gather(i):
                    off = i * 16
                    ii = idx_v[pl.ds(cix * _CH + off, 16)]
                    res_v[pl.ds(off, 16)] = plsc.load_gather(row_v, [ii])

                @pl.when(jnp.logical_not(is_lin))
                def _of():
                    pltpu.sync_copy(
                        res_v, fen_out.at[pl.ds(t * B + cix * _CH, _CH)])

                @pl.when(is_lin)
                def _ol():
                    pltpu.sync_copy(
                        res_v,
                        lin_out.at[pl.ds((t - nd) * B + cix * _CH, _CH)])

    call = pl.kernel(
        body,
        out_type=(
            jax.ShapeDtypeStruct((nf * D * B,), jnp.float32),
            jax.ShapeDtypeStruct((nf * B,), jnp.float32),
        ),
        mesh=plsc.VectorSubcoreMesh(core_axis_name="c", subcore_axis_name="s"),
        scratch_types=[
            pltpu.VMEM((V,), jnp.float32),
            pltpu.VMEM((B,), jnp.int32),
            pltpu.VMEM((_CH,), jnp.float32),
        ],
        compiler_params=pltpu.CompilerParams(
            use_tc_tiling_on_sc=False, needs_layout_passes=False),
    )
    return call


_CH = 4096
_G = (26,)


_BB = 2048


def _tc_body(fen_ref, lin_ref, den_ref,
             w1t_ref, b1_ref, w2t_ref, b2_ref,
             pt_ref, e_ref, s_ref, dw_ref, bias_ref, out_ref):
    x = fen_ref[...]
    lin_ref_all = lin_ref[...]
    h = jnp.dot(w1t_ref[...], x, preferred_element_type=jnp.float32)
    h = jnp.maximum(h + b1_ref[...], 0.0)
    h = jnp.dot(w2t_ref[...], h, preferred_element_type=jnp.float32)
    h = jnp.maximum(h + b2_ref[...], 0.0)
    logits = jnp.dot(pt_ref[...], h, preferred_element_type=jnp.float32)
    m = jnp.max(logits, axis=0, keepdims=True)
    e = jnp.exp(logits - m)
    mx = (float(F) / jnp.sum(e, axis=0, keepdims=True)) * e
    mx_exp = jnp.dot(e_ref[...], mx, preferred_element_type=jnp.float32)
    v = mx_exp * x
    sv = jnp.dot(s_ref[...], v, preferred_element_type=jnp.float32)
    fm = 0.5 * (jnp.sum(sv * sv, axis=0) - jnp.sum(v * v, axis=0))
    sp = jnp.sum(lin_ref_all * mx, axis=0)
    dn = jnp.sum(den_ref[...] * dw_ref[...], axis=0)
    out_ref[...] = fm + sp + dn + bias_ref[0, 0]


def kernel(sparse, dense, emb, lin_w, dense_w, W1, b1, W2, b2, P, bias):
    emb_t3 = emb.transpose(0, 2, 1)
    idx_t = sparse.T

    fens, lins = [], []
    f0 = 0
    for nf in _G:
        emb_g = emb_t3[f0:f0 + nf].reshape(nf * D * V)
        lin_g = lin_w[f0:f0 + nf].reshape(nf * V)
        idx_g = idx_t[f0:f0 + nf].reshape(nf * B)
        fen_f, lin_f = _make_sc_gather(nf)(emb_g, lin_g, idx_g)
        fens.append(fen_f.reshape(nf * D, B))
        lins.append(lin_f.reshape(nf, B))
        f0 += nf
    dense_t = dense.T

    expand_t = jnp.repeat(jnp.eye(F, dtype=jnp.float32), D, axis=0)
    reduce_t = jnp.tile(jnp.eye(D, dtype=jnp.float32), (1, F))

    out = pl.pallas_call(
        _tc_body,
        grid=(B // _BB,),
        in_specs=[
            pl.BlockSpec((FD, _BB), lambda i: (0, i)),
            pl.BlockSpec((F, _BB), lambda i: (0, i)),
            pl.BlockSpec((ND, _BB), lambda i: (0, i)),
            pl.BlockSpec((H1, FD), lambda i: (0, 0)),
            pl.BlockSpec((H1, 1), lambda i: (0, 0)),
            pl.BlockSpec((H2, H1), lambda i: (0, 0)),
            pl.BlockSpec((H2, 1), lambda i: (0, 0)),
            pl.BlockSpec((F, H2), lambda i: (0, 0)),
            pl.BlockSpec((FD, F), lambda i: (0, 0)),
            pl.BlockSpec((D, FD), lambda i: (0, 0)),
            pl.BlockSpec((ND, 1), lambda i: (0, 0)),
            pl.BlockSpec((1, 1), lambda i: (0, 0)),
        ],
        out_specs=pl.BlockSpec((_BB,), lambda i: (i,)),
        out_shape=jax.ShapeDtypeStruct((B,), jnp.float32),
    )(fens[0], lins[0], dense_t,
      W1.T, b1.reshape(H1, 1), W2.T, b2.reshape(H2, 1),
      P.T, expand_t, reduce_t, dense_w.reshape(ND, 1), bias.reshape(1, 1))
    return out
